# baseline (speedup 1.0000x reference)
import functools

import jax
import jax.numpy as jnp
from jax import lax
from jax.experimental import pallas as pl
from jax.experimental.pallas import tpu as pltpu
from jax.experimental.pallas import tpu_sc as plsc

E = 8
D = 1024
N = 4096
BLK = 256
G = N // BLK + E
P = G * BLK
DN = 256
NC, NS = 2, 16
NW = NC * NS


def _gather_rows(table, idx, chunk):
    M = idx.shape[0]
    C = table.shape[1]
    mpw = M // NW
    nch = mpw // chunk
    mesh = plsc.VectorSubcoreMesh(core_axis_name="c", subcore_axis_name="s")

    @functools.partial(
        pl.kernel,
        mesh=mesh,
        out_type=jax.ShapeDtypeStruct((M, C), table.dtype),
        scratch_types=[
            pltpu.VMEM((mpw,), jnp.int32),
            pltpu.VMEM((chunk, C), table.dtype),
            pltpu.VMEM((chunk, C), table.dtype),
            pltpu.SemaphoreType.DMA,
            pltpu.SemaphoreType.DMA,
        ],
    )
    def k(table_hbm, idx_hbm, out_hbm, idx_v, buf0, buf1, sem0, sem1):
        wid = lax.axis_index("s") * NC + lax.axis_index("c")
        base = wid * mpw
        pltpu.sync_copy(idx_hbm.at[pl.ds(base, mpw)], idx_v)
        bufs = (buf0, buf1)
        sems = (sem0, sem1)
        cps = [None, None]
        cps[0] = pltpu.async_copy(
            table_hbm.at[idx_v.at[pl.ds(0, chunk)]], buf0, sem0)
        for c in range(nch):
            s = c % 2
            if c + 1 < nch:
                cps[1 - s] = pltpu.async_copy(
                    table_hbm.at[idx_v.at[pl.ds((c + 1) * chunk, chunk)]],
                    bufs[1 - s], sems[1 - s])
            cps[s].wait()
            pltpu.sync_copy(bufs[s], out_hbm.at[pl.ds(base + c * chunk, chunk)])

    return k(table, idx)


def _grouped_matmul(xg, W, b, blk_e, nused, run_first, parity, next_e, has_next):
    def body(be_ref, nu_ref, rf_ref, par_ref, ne_ref, hn_ref,
             x_ref, w_hbm, b_ref, o_ref, wbuf, wbf, sems):
        g = pl.program_id(0)
        par = par_ref[g]

        @pl.when(g == 0)
        def _():
            pltpu.make_async_copy(w_hbm.at[be_ref[0]], wbuf.at[0],
                                  sems.at[0]).start()

        @pl.when((rf_ref[g] == 1) & (hn_ref[g] == 1))
        def _():
            pltpu.make_async_copy(w_hbm.at[ne_ref[g]], wbuf.at[1 - par],
                                  sems.at[1 - par]).start()

        @pl.when(rf_ref[g] == 1)
        def _():
            pltpu.make_async_copy(w_hbm.at[be_ref[g]], wbuf.at[par],
                                  sems.at[par]).wait()
            wbf[par] = wbuf[par].astype(jnp.bfloat16)

        @pl.when(g < nu_ref[0])
        def _():
            x = x_ref[...].astype(jnp.bfloat16)
            acc = jnp.dot(x, wbf[par], preferred_element_type=jnp.float32)
            o_ref[...] = acc + b_ref[0]

    grid_spec = pltpu.PrefetchScalarGridSpec(
        num_scalar_prefetch=6,
        grid=(G,),
        in_specs=[
            pl.BlockSpec((BLK, D), lambda g, *_: (g, 0)),
            pl.BlockSpec(memory_space=pl.ANY),
            pl.BlockSpec((1, 1, D), lambda g, be, *_: (be[g], 0, 0)),
        ],
        out_specs=pl.BlockSpec((BLK, D), lambda g, *_: (g, 0)),
        scratch_shapes=[
            pltpu.VMEM((2, D, D), jnp.float32),
            pltpu.VMEM((2, D, D), jnp.bfloat16),
            pltpu.SemaphoreType.DMA((2,)),
        ],
    )
    return pl.pallas_call(
        body,
        grid_spec=grid_spec,
        out_shape=jax.ShapeDtypeStruct((P, D), jnp.float32),
    )(blk_e, nused, run_first, parity, next_e, has_next,
      xg, W, b.reshape(E, 1, D))


def kernel(xs, mxs, actions, W, b):
    a = actions.astype(jnp.int32)
    oh = (a[:, None] == jnp.arange(E, dtype=jnp.int32)[None, :]).astype(jnp.int32)
    csum = jnp.cumsum(oh, axis=0)
    counts = csum[-1]
    rank = jnp.take_along_axis(csum - oh, a[:, None], axis=1)[:, 0]
    bpe = (counts + BLK - 1) // BLK
    starts = jnp.cumsum(bpe) - bpe
    pos = starts[a] * BLK + rank
    src = (jnp.arange(P, dtype=jnp.int32) % N).at[pos].set(
        jnp.arange(N, dtype=jnp.int32))
    gi = jnp.arange(G, dtype=jnp.int32)
    blk_e = jnp.clip(
        jnp.sum((gi[:, None] >= starts[None, :]).astype(jnp.int32), axis=1) - 1,
        0, E - 1)
    nused = jnp.sum(bpe, dtype=jnp.int32).reshape(1)
    chg = jnp.concatenate(
        [jnp.ones((1,), jnp.int32), (blk_e[1:] != blk_e[:-1]).astype(jnp.int32)])
    parity = ((jnp.cumsum(chg) - 1) % 2).astype(jnp.int32)
    arr = jnp.where(chg == 1, gi, G)
    nch = jnp.concatenate(
        [lax.cummin(arr, axis=0, reverse=True)[1:], jnp.full((1,), G, jnp.int32)])
    has_next = (nch < G).astype(jnp.int32)
    next_e = blk_e[jnp.minimum(nch, G - 1)]

    xg = _gather_rows(xs, src, 32)
    ys_sorted = _grouped_matmul(xg, W, b, blk_e, nused,
                                chg, parity, next_e, has_next)
    ys = _gather_rows(ys_sorted, pos, 32)
    return (ys, mxs, actions)


# speedup vs baseline: 1.0183x; 1.0183x over previous
import functools

import jax
import jax.numpy as jnp
from jax import lax
from jax.experimental import pallas as pl
from jax.experimental.pallas import tpu as pltpu
from jax.experimental.pallas import tpu_sc as plsc

E = 8
D = 1024
N = 4096
BLK = 256
G = N // BLK + E
P = G * BLK
DN = 256
NC, NS = 2, 16
NW = NC * NS


def _gather_rows(table, idx, chunk):
    M = idx.shape[0]
    C = table.shape[1]
    mpw = M // NW
    nch = mpw // chunk
    mesh = plsc.VectorSubcoreMesh(core_axis_name="c", subcore_axis_name="s")

    @functools.partial(
        pl.kernel,
        mesh=mesh,
        out_type=jax.ShapeDtypeStruct((M, C), table.dtype),
        scratch_types=[
            pltpu.VMEM((mpw,), jnp.int32),
            pltpu.VMEM((chunk, C), table.dtype),
            pltpu.VMEM((chunk, C), table.dtype),
            pltpu.SemaphoreType.DMA,
            pltpu.SemaphoreType.DMA,
        ],
    )
    def k(table_hbm, idx_hbm, out_hbm, idx_v, buf0, buf1, sem0, sem1):
        wid = lax.axis_index("s") * NC + lax.axis_index("c")
        base = wid * mpw
        pltpu.sync_copy(idx_hbm.at[pl.ds(base, mpw)], idx_v)
        bufs = (buf0, buf1)
        sems = (sem0, sem1)
        cps = [None, None]
        cps[0] = pltpu.async_copy(
            table_hbm.at[idx_v.at[pl.ds(0, chunk)]], buf0, sem0)
        for c in range(nch):
            s = c % 2
            if c + 1 < nch:
                cps[1 - s] = pltpu.async_copy(
                    table_hbm.at[idx_v.at[pl.ds((c + 1) * chunk, chunk)]],
                    bufs[1 - s], sems[1 - s])
            cps[s].wait()
            pltpu.sync_copy(bufs[s], out_hbm.at[pl.ds(base + c * chunk, chunk)])

    return k(table, idx)


def _grouped_matmul(xg, W, b, blk_e, nused, run_first, parity, next_e, has_next):
    def body(be_ref, nu_ref, rf_ref, par_ref, ne_ref, hn_ref,
             x_ref, w_hbm, b_ref, o_ref, wbuf, wbf, sems):
        g = pl.program_id(0)
        par = par_ref[g]

        @pl.when(g == 0)
        def _():
            pltpu.make_async_copy(w_hbm.at[be_ref[0]], wbuf.at[0],
                                  sems.at[0]).start()

        @pl.when((rf_ref[g] == 1) & (hn_ref[g] == 1))
        def _():
            pltpu.make_async_copy(w_hbm.at[ne_ref[g]], wbuf.at[1 - par],
                                  sems.at[1 - par]).start()

        @pl.when(rf_ref[g] == 1)
        def _():
            pltpu.make_async_copy(w_hbm.at[be_ref[g]], wbuf.at[par],
                                  sems.at[par]).wait()
            wbf[par] = wbuf[par].astype(jnp.bfloat16)

        @pl.when(g < nu_ref[0])
        def _():
            x = x_ref[...].astype(jnp.bfloat16)
            acc = jnp.dot(x, wbf[par], preferred_element_type=jnp.float32)
            o_ref[...] = acc + b_ref[0]

    grid_spec = pltpu.PrefetchScalarGridSpec(
        num_scalar_prefetch=6,
        grid=(G,),
        in_specs=[
            pl.BlockSpec((BLK, D), lambda g, *_: (g, 0)),
            pl.BlockSpec(memory_space=pl.ANY),
            pl.BlockSpec((1, 1, D), lambda g, be, *_: (be[g], 0, 0)),
        ],
        out_specs=pl.BlockSpec((BLK, D), lambda g, *_: (g, 0)),
        scratch_shapes=[
            pltpu.VMEM((2, D, D), jnp.float32),
            pltpu.VMEM((2, D, D), jnp.bfloat16),
            pltpu.SemaphoreType.DMA((2,)),
        ],
    )
    return pl.pallas_call(
        body,
        grid_spec=grid_spec,
        out_shape=jax.ShapeDtypeStruct((P, D), jnp.float32),
    )(blk_e, nused, run_first, parity, next_e, has_next,
      xg, W, b.reshape(E, 1, D))


def kernel(xs, mxs, actions, W, b):
    a = actions.astype(jnp.int32)
    oh = (a[:, None] == jnp.arange(E, dtype=jnp.int32)[None, :]).astype(jnp.int16)
    csum = jnp.cumsum(oh, axis=0)
    counts = csum[-1].astype(jnp.int32)
    rank = jnp.take_along_axis(csum, a[:, None], axis=1)[:, 0].astype(jnp.int32) - 1
    bpe = (counts + BLK - 1) // BLK
    starts = jnp.cumsum(bpe) - bpe
    pos = starts[a] * BLK + rank
    src = (jnp.arange(P, dtype=jnp.int32) % N).at[pos].set(
        jnp.arange(N, dtype=jnp.int32))
    gi = jnp.arange(G, dtype=jnp.int32)
    blk_e = jnp.clip(
        jnp.sum((gi[:, None] >= starts[None, :]).astype(jnp.int32), axis=1) - 1,
        0, E - 1)
    nused = jnp.sum(bpe, dtype=jnp.int32).reshape(1)
    chg = jnp.concatenate(
        [jnp.ones((1,), jnp.int32), (blk_e[1:] != blk_e[:-1]).astype(jnp.int32)])
    parity = ((jnp.cumsum(chg) - 1) % 2).astype(jnp.int32)
    arr = jnp.where(chg == 1, gi, G)
    nch = jnp.concatenate(
        [lax.cummin(arr, axis=0, reverse=True)[1:], jnp.full((1,), G, jnp.int32)])
    has_next = (nch < G).astype(jnp.int32)
    next_e = blk_e[jnp.minimum(nch, G - 1)]

    xg = _gather_rows(xs, src, 32)
    ys_sorted = _grouped_matmul(xg, W, b, blk_e, nused,
                                chg, parity, next_e, has_next)
    ys = _gather_rows(ys_sorted, pos, 32)
    return (ys, mxs, actions)
